# baseline (speedup 1.0000x reference)
import functools

import jax
import jax.numpy as jnp
from jax import lax
from jax.experimental import pallas as pl
from jax.experimental.pallas import tpu as pltpu
from jax.experimental.pallas import tpu_sc as plsc

N = 100000
E = 3200000
D = 16
NC = 2
NS = 16
NW = NC * NS
LANES = 16

NP = 100352
CHUNK = 128
NBUF = 8
EP = 784 * NW * CHUNK
EW = EP // NW
CPW = EW // CHUNK
ITERS = CPW // NBUF
NSL = NP // NS
NWSL = NP // NW
PCH = 448
SGRP = 4
C0 = CPW
C1 = CPW

_mesh = plsc.VectorSubcoreMesh(
    core_axis_name="c", subcore_axis_name="s", num_cores=NC, num_subcores=NS
)
_cparams = pltpu.CompilerParams(
    use_tc_tiling_on_sc=False, needs_layout_passes=False
)

_f32 = jnp.float32
_i32 = jnp.int32


def _iota16():
  return lax.iota(_i32, LANES)


def _splat16(v):
  return jnp.full((LANES,), v, _i32)


def _bcast_lane(vec, n):
  idx = jnp.full((LANES,), n, _i32)
  return jnp.take_along_axis(vec, idx, axis=0, mode="promise_in_bounds")


def _rsqrt16(x):
  i = lax.bitcast_convert_type(x, _i32)
  i = jnp.int32(0x5F3759DF) - lax.shift_right_arithmetic(i, 1)
  y = lax.bitcast_convert_type(i, _f32)
  for _ in range(3):
    y = y * (1.5 - 0.5 * x * y * y)
  return y


def _wid():
  return lax.axis_index("s") * NC + lax.axis_index("c")


def _edge_share():
  cid = lax.axis_index("c")
  sid = lax.axis_index("s")
  row0 = sid * (C0 + C1) + jnp.where(cid == 0, 0, C0)
  cnt = jnp.where(cid == 0, C0, C1)
  return row0, cnt


def _deg_pass_body(dst_hbm, d0_hbm, d1_hbm, didx, ones_v, zbuf, spdeg,
                   isem, ssem):
  cid = lax.axis_index("c")
  sid = lax.axis_index("s")
  wid = _wid()

  one16 = jnp.ones((LANES,), _f32)
  zero16 = jnp.zeros((LANES,), _f32)
  for r in range(CHUNK // LANES):
    ones_v[pl.ds(r * LANES, LANES)] = one16
  for r in range(PCH // LANES):
    zbuf[pl.ds(r * LANES, LANES)] = zero16
  zoff = sid * NSL
  @pl.loop(0, NSL // PCH)
  def _zero(k):
    pltpu.sync_copy(zbuf, spdeg.at[pl.ds(zoff + k * PCH, PCH)])

  plsc.subcore_barrier()

  row0, cnt = _edge_share()

  @pl.loop(0, cnt // NBUF)
  def _iter(g):
    r = row0 + g * NBUF
    dj = pltpu.async_copy(dst_hbm.at[pl.ds(r, NBUF), :], didx, isem.at[0])
    dj.wait()
    sds = []
    for b in range(NBUF):
      sds.append(pltpu.async_copy(
          ones_v, spdeg.at[didx.at[b]], ssem.at[b], add=True))
    for b in range(NBUF):
      sds[b].wait()

  plsc.subcore_barrier()

  off = sid * NSL
  @pl.when(cid == 0)
  def _():
    pltpu.sync_copy(spdeg.at[pl.ds(off, NSL)], d0_hbm.at[pl.ds(off, NSL)])
  @pl.when(cid == 1)
  def _():
    pltpu.sync_copy(spdeg.at[pl.ds(off, NSL)], d1_hbm.at[pl.ds(off, NSL)])


_deg_pass = pl.kernel(
    _deg_pass_body,
    out_type=(jax.ShapeDtypeStruct((NP,), _f32),
              jax.ShapeDtypeStruct((NP,), _f32)),
    mesh=_mesh,
    compiler_params=_cparams,
    scratch_types=[
        pltpu.VMEM((NBUF, CHUNK), _i32),
        pltpu.VMEM((CHUNK,), _f32),
        pltpu.VMEM((PCH,), _f32),
        pltpu.VMEM_SHARED((NP,), _f32),
        pltpu.SemaphoreType.DMA((1,)),
        pltpu.SemaphoreType.DMA((NBUF,)),
    ],
)


def _edge_pass_body(y_hbm, src_hbm, dst_hbm, a0_hbm, a1_hbm,
                    sidx, didx, rows, zbuf, spacc, isem, gsem, ssem):
  cid = lax.axis_index("c")
  sid = lax.axis_index("s")
  wid = _wid()

  z16 = jnp.zeros((LANES,), _f32)
  for r in range(CHUNK):
    zbuf[r, :] = z16
  zoff = sid * NSL
  @pl.loop(0, NSL // CHUNK)
  def _zero(k):
    pltpu.sync_copy(zbuf, spacc.at[pl.ds(zoff + k * CHUNK, CHUNK), :])

  plsc.subcore_barrier()

  row0, cnt = _edge_share()

  @pl.loop(0, cnt // NBUF)
  def _iter(g):
    r = row0 + g * NBUF
    di = pltpu.async_copy(src_hbm.at[pl.ds(r, NBUF), :], sidx, isem.at[0])
    dj = pltpu.async_copy(dst_hbm.at[pl.ds(r, NBUF), :], didx, isem.at[1])
    di.wait()
    dj.wait()
    gds = []
    for b in range(NBUF):
      gds.append(pltpu.async_copy(
          y_hbm.at[sidx.at[b]], rows.at[pl.ds(b * CHUNK, CHUNK), :],
          gsem.at[b]))
    sds = []
    for b in range(NBUF):
      gds[b].wait()
      sds.append(pltpu.async_copy(
          rows.at[pl.ds(b * CHUNK, CHUNK), :], spacc.at[didx.at[b]],
          ssem.at[b], add=True))
    for b in range(NBUF):
      sds[b].wait()

  plsc.subcore_barrier()

  off = sid * NSL
  @pl.when(cid == 0)
  def _():
    pltpu.sync_copy(spacc.at[pl.ds(off, NSL), :], a0_hbm.at[pl.ds(off, NSL), :])
  @pl.when(cid == 1)
  def _():
    pltpu.sync_copy(spacc.at[pl.ds(off, NSL), :], a1_hbm.at[pl.ds(off, NSL), :])


_edge_pass = pl.kernel(
    _edge_pass_body,
    out_type=(jax.ShapeDtypeStruct((NP, D), _f32),
              jax.ShapeDtypeStruct((NP, D), _f32)),
    mesh=_mesh,
    compiler_params=_cparams,
    scratch_types=[
        pltpu.VMEM((NBUF, CHUNK), _i32),
        pltpu.VMEM((NBUF, CHUNK), _i32),
        pltpu.VMEM((NBUF * CHUNK, D), _f32),
        pltpu.VMEM((CHUNK, D), _f32),
        pltpu.VMEM_SHARED((NP, D), _f32),
        pltpu.SemaphoreType.DMA((2,)),
        pltpu.SemaphoreType.DMA((NBUF,)),
        pltpu.SemaphoreType.DMA((NBUF,)),
    ],
)


def _prep_body(d0_hbm, d1_hbm, emb_hbm, arow_hbm,
               dinv16_hbm, y_hbm, out_hbm,
               v_d0, v_d1, v_emb, v_y, v_out, v_d16, v_arow):
  wid = _wid()
  pltpu.sync_copy(arow_hbm, v_arow)
  a = v_arow[...]

  @pl.loop(0, NWSL // PCH)
  def _chunk(c):
    base = wid * NWSL + c * PCH
    pltpu.sync_copy(d0_hbm.at[pl.ds(base, PCH)], v_d0)
    pltpu.sync_copy(d1_hbm.at[pl.ds(base, PCH)], v_d1)
    pltpu.sync_copy(emb_hbm.at[pl.ds(base, PCH), :], v_emb)
    for g in range(PCH // LANES):
      deg = (v_d0[pl.ds(g * LANES, LANES)]
             + v_d1[pl.ds(g * LANES, LANES)])
      degc = jnp.maximum(deg, 1.0)
      dv = jnp.where(deg > 0.0, _rsqrt16(degc), 0.0)
      for n in range(LANES):
        r = v_emb[g * LANES + n, :]
        bv = _bcast_lane(dv, n)
        v_d16[g * LANES + n, :] = bv
        v_y[g * LANES + n, :] = bv * r
        v_out[g * LANES + n, :] = a * r
    pltpu.sync_copy(v_d16, dinv16_hbm.at[pl.ds(base, PCH), :])
    pltpu.sync_copy(v_y, y_hbm.at[pl.ds(base, PCH), :])
    pltpu.sync_copy(v_out, out_hbm.at[pl.ds(base, PCH), :])


_prep = pl.kernel(
    _prep_body,
    out_type=(jax.ShapeDtypeStruct((NP, D), _f32),
              jax.ShapeDtypeStruct((NP, D), _f32),
              jax.ShapeDtypeStruct((NP, D), _f32)),
    mesh=_mesh,
    compiler_params=_cparams,
    scratch_types=[
        pltpu.VMEM((PCH,), _f32),
        pltpu.VMEM((PCH,), _f32),
        pltpu.VMEM((PCH, D), _f32),
        pltpu.VMEM((PCH, D), _f32),
        pltpu.VMEM((PCH, D), _f32),
        pltpu.VMEM((PCH, D), _f32),
        pltpu.VMEM((LANES,), _f32),
    ],
)


NPF = NP * D // 128
MBLK = 896


def _merge_tc_body(a0, a1, op, dv, al, y, out):
  x = dv[...] * (a0[...] + a1[...])
  y[...] = dv[...] * x
  out[...] = op[...] + al[...] * x


_merge = pl.pallas_call(
    _merge_tc_body,
    grid=(NPF // MBLK,),
    in_specs=[pl.BlockSpec((MBLK, 128), lambda i: (i, 0))] * 4
    + [pl.BlockSpec((1, 128), lambda i: (0, 0))],
    out_specs=[pl.BlockSpec((MBLK, 128), lambda i: (i, 0))] * 2,
    out_shape=(jax.ShapeDtypeStruct((NPF, 128), _f32),
               jax.ShapeDtypeStruct((NPF, 128), _f32)),
)


def _dot_chunk(srows, drows, v_res, cbase):
  @pl.loop(0, CHUNK // LANES, unroll=2)
  def _grp(g2):
    base16 = cbase + g2 * LANES
    v_res[pl.ds(base16, LANES)] = jnp.zeros((LANES,), _f32)
    for n in range(LANES):
      prod = srows[base16 + n, :] * drows[base16 + n, :]
      plsc.addupdate_scatter(v_res, [_splat16(0) + (base16 + n)], prod)


def _edge_dot_body(out_hbm, src_hbm, dst_hbm, res_hbm,
                   sidx, didx, srows, drows, v_res, isem, gsem):
  row0, cnt = _edge_share()

  @pl.loop(0, cnt // (2 * NBUF))
  def _iter(g):
    rA = row0 + g * (2 * NBUF)
    rB = rA + NBUF
    diA = pltpu.async_copy(
        src_hbm.at[pl.ds(rA, NBUF), :], sidx.at[pl.ds(0, NBUF), :], isem.at[0])
    djA = pltpu.async_copy(
        dst_hbm.at[pl.ds(rA, NBUF), :], didx.at[pl.ds(0, NBUF), :], isem.at[1])
    diA.wait()
    djA.wait()
    gds = []
    for b in range(NBUF):
      gds.append((
          pltpu.async_copy(
              out_hbm.at[sidx.at[b]], srows.at[pl.ds(b * CHUNK, CHUNK), :],
              gsem.at[b]),
          pltpu.async_copy(
              out_hbm.at[didx.at[b]], drows.at[pl.ds(b * CHUNK, CHUNK), :],
              gsem.at[b])))
    diB = pltpu.async_copy(
        src_hbm.at[pl.ds(rB, NBUF), :], sidx.at[pl.ds(NBUF, NBUF), :],
        isem.at[2])
    djB = pltpu.async_copy(
        dst_hbm.at[pl.ds(rB, NBUF), :], didx.at[pl.ds(NBUF, NBUF), :],
        isem.at[3])
    diB.wait()
    djB.wait()
    for b in range(NBUF):
      k = NBUF + b
      gds.append((
          pltpu.async_copy(
              out_hbm.at[sidx.at[k]], srows.at[pl.ds(k * CHUNK, CHUNK), :],
              gsem.at[k]),
          pltpu.async_copy(
              out_hbm.at[didx.at[k]], drows.at[pl.ds(k * CHUNK, CHUNK), :],
              gsem.at[k])))
    for k in range(2 * NBUF):
      gds[k][0].wait()
      gds[k][1].wait()
      _dot_chunk(srows, drows, v_res, k * CHUNK)

    pltpu.sync_copy(v_res, res_hbm.at[pl.ds(rA * CHUNK, 2 * NBUF * CHUNK)])


_edge_dot = pl.kernel(
    _edge_dot_body,
    out_type=jax.ShapeDtypeStruct((EP,), _f32),
    mesh=_mesh,
    compiler_params=_cparams,
    scratch_types=[
        pltpu.VMEM((2 * NBUF, CHUNK), _i32),
        pltpu.VMEM((2 * NBUF, CHUNK), _i32),
        pltpu.VMEM((2 * NBUF * CHUNK, D), _f32),
        pltpu.VMEM((2 * NBUF * CHUNK, D), _f32),
        pltpu.VMEM((2 * NBUF * CHUNK,), _f32),
        pltpu.SemaphoreType.DMA((4,)),
        pltpu.SemaphoreType.DMA((2 * NBUF,)),
    ],
)


@jax.jit
def kernel(edge_index, emb, alpha):
  nlayers = alpha.shape[0] - 1

  pad = jnp.full((EP - E,), N, _i32)
  src2 = jnp.concatenate([edge_index[0], pad]).reshape(EP // CHUNK, CHUNK)
  dst2 = jnp.concatenate([edge_index[1], pad]).reshape(EP // CHUNK, CHUNK)

  embp = jnp.zeros((NP, D), _f32).at[:N, :].set(emb)
  a16 = jnp.broadcast_to(alpha[:, None], (alpha.shape[0], LANES))

  d0, d1 = _deg_pass(dst2)
  dinv16, y, out = _prep(d0, d1, embp, a16[0])
  dvf = dinv16.reshape(NPF, 128)
  outf = out.reshape(NPF, 128)
  for l in range(1, nlayers + 1):
    a0, a1 = _edge_pass(y, src2, dst2)
    alf = jnp.broadcast_to(alpha[l], (1, 128))
    yf, outf = _merge(a0.reshape(NPF, 128), a1.reshape(NPF, 128),
                      outf, dvf, alf)
    y = yf.reshape(NP, D)
  res = _edge_dot(outf.reshape(NP, D), src2, dst2)
  return res[:E]


# speedup vs baseline: 1.3499x; 1.3499x over previous
import functools

import jax
import jax.numpy as jnp
from jax import lax
from jax.experimental import pallas as pl
from jax.experimental.pallas import tpu as pltpu
from jax.experimental.pallas import tpu_sc as plsc

N = 100000
E = 3200000
D = 16
NC = 2
NS = 16
NW = NC * NS
LANES = 16

NP = 100352
CHUNK = 128
NBUF = 8
EP = 784 * NW * CHUNK
EW = EP // NW
CPW = EW // CHUNK
ITERS = CPW // NBUF
NSL = NP // NS
NWSL = NP // NW
PCH = 448
SGRP = 4
C0 = CPW
C1 = CPW

_mesh = plsc.VectorSubcoreMesh(
    core_axis_name="c", subcore_axis_name="s", num_cores=NC, num_subcores=NS
)
_cparams = pltpu.CompilerParams(
    use_tc_tiling_on_sc=False, needs_layout_passes=False
)

_f32 = jnp.float32
_i32 = jnp.int32


def _iota16():
  return lax.iota(_i32, LANES)


def _splat16(v):
  return jnp.full((LANES,), v, _i32)


def _bcast_lane(vec, n):
  idx = jnp.full((LANES,), n, _i32)
  return jnp.take_along_axis(vec, idx, axis=0, mode="promise_in_bounds")


def _rsqrt16(x):
  i = lax.bitcast_convert_type(x, _i32)
  i = jnp.int32(0x5F3759DF) - lax.shift_right_arithmetic(i, 1)
  y = lax.bitcast_convert_type(i, _f32)
  for _ in range(3):
    y = y * (1.5 - 0.5 * x * y * y)
  return y


def _wid():
  return lax.axis_index("s") * NC + lax.axis_index("c")


def _edge_share():
  cid = lax.axis_index("c")
  sid = lax.axis_index("s")
  row0 = sid * (C0 + C1) + jnp.where(cid == 0, 0, C0)
  cnt = jnp.where(cid == 0, C0, C1)
  return row0, cnt


def _deg_pass_body(dst_hbm, d0_hbm, d1_hbm, didx, ones_v, zbuf, spdeg,
                   isem, ssem):
  cid = lax.axis_index("c")
  sid = lax.axis_index("s")
  wid = _wid()

  one16 = jnp.ones((LANES,), _f32)
  zero16 = jnp.zeros((LANES,), _f32)
  for r in range(CHUNK // LANES):
    ones_v[pl.ds(r * LANES, LANES)] = one16
  for r in range(PCH // LANES):
    zbuf[pl.ds(r * LANES, LANES)] = zero16
  zoff = sid * NSL
  @pl.loop(0, NSL // PCH)
  def _zero(k):
    pltpu.sync_copy(zbuf, spdeg.at[pl.ds(zoff + k * PCH, PCH)])

  plsc.subcore_barrier()

  row0, cnt = _edge_share()

  @pl.loop(0, cnt // NBUF)
  def _iter(g):
    r = row0 + g * NBUF
    dj = pltpu.async_copy(dst_hbm.at[pl.ds(r, NBUF), :], didx, isem.at[0])
    dj.wait()
    sds = []
    for b in range(NBUF):
      sds.append(pltpu.async_copy(
          ones_v, spdeg.at[didx.at[b]], ssem.at[b], add=True))
    for b in range(NBUF):
      sds[b].wait()

  plsc.subcore_barrier()

  off = sid * NSL
  @pl.when(cid == 0)
  def _():
    pltpu.sync_copy(spdeg.at[pl.ds(off, NSL)], d0_hbm.at[pl.ds(off, NSL)])
  @pl.when(cid == 1)
  def _():
    pltpu.sync_copy(spdeg.at[pl.ds(off, NSL)], d1_hbm.at[pl.ds(off, NSL)])


_deg_pass = pl.kernel(
    _deg_pass_body,
    out_type=(jax.ShapeDtypeStruct((NP,), _f32),
              jax.ShapeDtypeStruct((NP,), _f32)),
    mesh=_mesh,
    compiler_params=_cparams,
    scratch_types=[
        pltpu.VMEM((NBUF, CHUNK), _i32),
        pltpu.VMEM((CHUNK,), _f32),
        pltpu.VMEM((PCH,), _f32),
        pltpu.VMEM_SHARED((NP,), _f32),
        pltpu.SemaphoreType.DMA((1,)),
        pltpu.SemaphoreType.DMA((NBUF,)),
    ],
)


def _edge_pass_body(y_hbm, src_hbm, dst_hbm, a0_hbm, a1_hbm,
                    sidx, didx, rows, zbuf, spacc, isem, gsem, ssem):
  cid = lax.axis_index("c")
  sid = lax.axis_index("s")
  wid = _wid()

  z16 = jnp.zeros((LANES,), _f32)
  for r in range(CHUNK):
    zbuf[r, :] = z16
  zoff = sid * NSL
  @pl.loop(0, NSL // CHUNK)
  def _zero(k):
    pltpu.sync_copy(zbuf, spacc.at[pl.ds(zoff + k * CHUNK, CHUNK), :])

  plsc.subcore_barrier()

  row0, cnt = _edge_share()

  @pl.loop(0, cnt // NBUF)
  def _iter(g):
    r = row0 + g * NBUF
    di = pltpu.async_copy(src_hbm.at[pl.ds(r, NBUF), :], sidx, isem.at[0])
    dj = pltpu.async_copy(dst_hbm.at[pl.ds(r, NBUF), :], didx, isem.at[1])
    di.wait()
    dj.wait()
    gds = []
    for b in range(NBUF):
      gds.append(pltpu.async_copy(
          y_hbm.at[sidx.at[b]], rows.at[pl.ds(b * CHUNK, CHUNK), :],
          gsem.at[b]))
    sds = []
    for b in range(NBUF):
      gds[b].wait()
      sds.append(pltpu.async_copy(
          rows.at[pl.ds(b * CHUNK, CHUNK), :], spacc.at[didx.at[b]],
          ssem.at[b], add=True))
    for b in range(NBUF):
      sds[b].wait()

  plsc.subcore_barrier()

  off = sid * NSL
  @pl.when(cid == 0)
  def _():
    pltpu.sync_copy(spacc.at[pl.ds(off, NSL), :], a0_hbm.at[pl.ds(off, NSL), :])
  @pl.when(cid == 1)
  def _():
    pltpu.sync_copy(spacc.at[pl.ds(off, NSL), :], a1_hbm.at[pl.ds(off, NSL), :])


_edge_pass = pl.kernel(
    _edge_pass_body,
    out_type=(jax.ShapeDtypeStruct((NP, D), _f32),
              jax.ShapeDtypeStruct((NP, D), _f32)),
    mesh=_mesh,
    compiler_params=_cparams,
    scratch_types=[
        pltpu.VMEM((NBUF, CHUNK), _i32),
        pltpu.VMEM((NBUF, CHUNK), _i32),
        pltpu.VMEM((NBUF * CHUNK, D), _f32),
        pltpu.VMEM((CHUNK, D), _f32),
        pltpu.VMEM_SHARED((NP, D), _f32),
        pltpu.SemaphoreType.DMA((2,)),
        pltpu.SemaphoreType.DMA((NBUF,)),
        pltpu.SemaphoreType.DMA((NBUF,)),
    ],
)


def _prep_body(d0_hbm, d1_hbm, emb_hbm, arow_hbm,
               dinv16_hbm, y_hbm, out_hbm,
               v_d0, v_d1, v_emb, v_y, v_out, v_d16, v_arow):
  wid = _wid()
  pltpu.sync_copy(arow_hbm, v_arow)
  a = v_arow[...]

  @pl.loop(0, NWSL // PCH)
  def _chunk(c):
    base = wid * NWSL + c * PCH
    pltpu.sync_copy(d0_hbm.at[pl.ds(base, PCH)], v_d0)
    pltpu.sync_copy(d1_hbm.at[pl.ds(base, PCH)], v_d1)
    pltpu.sync_copy(emb_hbm.at[pl.ds(base, PCH), :], v_emb)
    for g in range(PCH // LANES):
      deg = (v_d0[pl.ds(g * LANES, LANES)]
             + v_d1[pl.ds(g * LANES, LANES)])
      degc = jnp.maximum(deg, 1.0)
      dv = jnp.where(deg > 0.0, _rsqrt16(degc), 0.0)
      for n in range(LANES):
        r = v_emb[g * LANES + n, :]
        bv = _bcast_lane(dv, n)
        v_d16[g * LANES + n, :] = bv
        v_y[g * LANES + n, :] = bv * r
        v_out[g * LANES + n, :] = a * r
    pltpu.sync_copy(v_d16, dinv16_hbm.at[pl.ds(base, PCH), :])
    pltpu.sync_copy(v_y, y_hbm.at[pl.ds(base, PCH), :])
    pltpu.sync_copy(v_out, out_hbm.at[pl.ds(base, PCH), :])


_prep = pl.kernel(
    _prep_body,
    out_type=(jax.ShapeDtypeStruct((NP, D), _f32),
              jax.ShapeDtypeStruct((NP, D), _f32),
              jax.ShapeDtypeStruct((NP, D), _f32)),
    mesh=_mesh,
    compiler_params=_cparams,
    scratch_types=[
        pltpu.VMEM((PCH,), _f32),
        pltpu.VMEM((PCH,), _f32),
        pltpu.VMEM((PCH, D), _f32),
        pltpu.VMEM((PCH, D), _f32),
        pltpu.VMEM((PCH, D), _f32),
        pltpu.VMEM((PCH, D), _f32),
        pltpu.VMEM((LANES,), _f32),
    ],
)


NPF = NP * D // 128
MBLK = 896


def _merge_tc_body(a0, a1, op, dv, al, y, out):
  x = dv[...] * (a0[...] + a1[...])
  y[...] = dv[...] * x
  out[...] = op[...] + al[...] * x


_merge = pl.pallas_call(
    _merge_tc_body,
    grid=(NPF // MBLK,),
    in_specs=[pl.BlockSpec((MBLK, 128), lambda i: (i, 0))] * 4
    + [pl.BlockSpec((1, 128), lambda i: (0, 0))],
    out_specs=[pl.BlockSpec((MBLK, 128), lambda i: (i, 0))] * 2,
    out_shape=(jax.ShapeDtypeStruct((NPF, 128), _f32),
               jax.ShapeDtypeStruct((NPF, 128), _f32)),
)


def _dot_chunk(srows, drows, v_res, cbase):
  lane15 = _iota16() == (LANES - 1)

  @pl.loop(0, CHUNK // LANES, unroll=2)
  def _grp(g2):
    base16 = cbase + g2 * LANES
    for n in range(LANES):
      prod = srows[base16 + n, :] * drows[base16 + n, :]
      cs = plsc.cumsum(prod)
      plsc.store_scatter(v_res, [_splat16(0) + (base16 + n)], cs, mask=lane15)


def _edge_dot_body(out_hbm, src_hbm, dst_hbm, res_hbm,
                   sidx, didx, srows, drows, v_res, isem, gsem):
  row0, cnt = _edge_share()

  @pl.loop(0, cnt // (2 * NBUF))
  def _iter(g):
    rA = row0 + g * (2 * NBUF)
    rB = rA + NBUF
    diA = pltpu.async_copy(
        src_hbm.at[pl.ds(rA, NBUF), :], sidx.at[pl.ds(0, NBUF), :], isem.at[0])
    djA = pltpu.async_copy(
        dst_hbm.at[pl.ds(rA, NBUF), :], didx.at[pl.ds(0, NBUF), :], isem.at[1])
    diA.wait()
    djA.wait()
    gds = []
    for b in range(NBUF):
      gds.append((
          pltpu.async_copy(
              out_hbm.at[sidx.at[b]], srows.at[pl.ds(b * CHUNK, CHUNK), :],
              gsem.at[b]),
          pltpu.async_copy(
              out_hbm.at[didx.at[b]], drows.at[pl.ds(b * CHUNK, CHUNK), :],
              gsem.at[b])))
    diB = pltpu.async_copy(
        src_hbm.at[pl.ds(rB, NBUF), :], sidx.at[pl.ds(NBUF, NBUF), :],
        isem.at[2])
    djB = pltpu.async_copy(
        dst_hbm.at[pl.ds(rB, NBUF), :], didx.at[pl.ds(NBUF, NBUF), :],
        isem.at[3])
    diB.wait()
    djB.wait()
    for b in range(NBUF):
      k = NBUF + b
      gds.append((
          pltpu.async_copy(
              out_hbm.at[sidx.at[k]], srows.at[pl.ds(k * CHUNK, CHUNK), :],
              gsem.at[k]),
          pltpu.async_copy(
              out_hbm.at[didx.at[k]], drows.at[pl.ds(k * CHUNK, CHUNK), :],
              gsem.at[k])))
    for k in range(2 * NBUF):
      gds[k][0].wait()
      gds[k][1].wait()
      _dot_chunk(srows, drows, v_res, k * CHUNK)

    pltpu.sync_copy(v_res, res_hbm.at[pl.ds(rA * CHUNK, 2 * NBUF * CHUNK)])


_edge_dot = pl.kernel(
    _edge_dot_body,
    out_type=jax.ShapeDtypeStruct((EP,), _f32),
    mesh=_mesh,
    compiler_params=_cparams,
    scratch_types=[
        pltpu.VMEM((2 * NBUF, CHUNK), _i32),
        pltpu.VMEM((2 * NBUF, CHUNK), _i32),
        pltpu.VMEM((2 * NBUF * CHUNK, D), _f32),
        pltpu.VMEM((2 * NBUF * CHUNK, D), _f32),
        pltpu.VMEM((2 * NBUF * CHUNK,), _f32),
        pltpu.SemaphoreType.DMA((4,)),
        pltpu.SemaphoreType.DMA((2 * NBUF,)),
    ],
)


@jax.jit
def kernel(edge_index, emb, alpha):
  nlayers = alpha.shape[0] - 1

  pad = jnp.full((EP - E,), N, _i32)
  src2 = jnp.concatenate([edge_index[0], pad]).reshape(EP // CHUNK, CHUNK)
  dst2 = jnp.concatenate([edge_index[1], pad]).reshape(EP // CHUNK, CHUNK)

  embp = jnp.zeros((NP, D), _f32).at[:N, :].set(emb)
  a16 = jnp.broadcast_to(alpha[:, None], (alpha.shape[0], LANES))

  d0, d1 = _deg_pass(dst2)
  dinv16, y, out = _prep(d0, d1, embp, a16[0])
  dvf = dinv16.reshape(NPF, 128)
  outf = out.reshape(NPF, 128)
  for l in range(1, nlayers + 1):
    a0, a1 = _edge_pass(y, src2, dst2)
    alf = jnp.broadcast_to(alpha[l], (1, 128))
    yf, outf = _merge(a0.reshape(NPF, 128), a1.reshape(NPF, 128),
                      outf, dvf, alf)
    y = yf.reshape(NP, D)
  res = _edge_dot(outf.reshape(NP, D), src2, dst2)
  return res[:E]


# speedup vs baseline: 2.0731x; 1.5357x over previous
import functools

import jax
import jax.numpy as jnp
from jax import lax
from jax.experimental import pallas as pl
from jax.experimental.pallas import tpu as pltpu
from jax.experimental.pallas import tpu_sc as plsc

N = 100000
E = 3200000
D = 16
NC = 2
NS = 16
NW = NC * NS
LANES = 16

NP = 100352
CHUNK = 128
NBUF = 8
EP = 784 * NW * CHUNK
EW = EP // NW
CPW = EW // CHUNK
ITERS = CPW // NBUF
NSL = NP // NS
NWSL = NP // NW
PCH = 448
SGRP = 4
C0 = CPW
C1 = CPW

_mesh = plsc.VectorSubcoreMesh(
    core_axis_name="c", subcore_axis_name="s", num_cores=NC, num_subcores=NS
)
_cparams = pltpu.CompilerParams(
    use_tc_tiling_on_sc=False, needs_layout_passes=False
)

_f32 = jnp.float32
_i32 = jnp.int32


def _iota16():
  return lax.iota(_i32, LANES)


def _splat16(v):
  return jnp.full((LANES,), v, _i32)


def _bcast_lane(vec, n):
  idx = jnp.full((LANES,), n, _i32)
  return jnp.take_along_axis(vec, idx, axis=0, mode="promise_in_bounds")


def _rsqrt16(x):
  i = lax.bitcast_convert_type(x, _i32)
  i = jnp.int32(0x5F3759DF) - lax.shift_right_arithmetic(i, 1)
  y = lax.bitcast_convert_type(i, _f32)
  for _ in range(3):
    y = y * (1.5 - 0.5 * x * y * y)
  return y


def _wid():
  return lax.axis_index("s") * NC + lax.axis_index("c")


def _edge_share():
  cid = lax.axis_index("c")
  sid = lax.axis_index("s")
  row0 = sid * (C0 + C1) + jnp.where(cid == 0, 0, C0)
  cnt = jnp.where(cid == 0, C0, C1)
  return row0, cnt


def _deg_pass_body(dst_hbm, d0_hbm, d1_hbm, didx, ones_v, zbuf, spdeg,
                   isem, ssem):
  cid = lax.axis_index("c")
  sid = lax.axis_index("s")
  wid = _wid()

  one16 = jnp.ones((LANES,), _f32)
  zero16 = jnp.zeros((LANES,), _f32)
  for r in range(CHUNK // LANES):
    ones_v[pl.ds(r * LANES, LANES)] = one16
  for r in range(PCH // LANES):
    zbuf[pl.ds(r * LANES, LANES)] = zero16
  zoff = sid * NSL
  @pl.loop(0, NSL // PCH)
  def _zero(k):
    pltpu.sync_copy(zbuf, spdeg.at[pl.ds(zoff + k * PCH, PCH)])

  plsc.subcore_barrier()

  row0, cnt = _edge_share()

  @pl.loop(0, cnt // NBUF)
  def _iter(g):
    r = row0 + g * NBUF
    dj = pltpu.async_copy(dst_hbm.at[pl.ds(r, NBUF), :], didx, isem.at[0])
    dj.wait()
    sds = []
    for b in range(NBUF):
      sds.append(pltpu.async_copy(
          ones_v, spdeg.at[didx.at[b]], ssem.at[b], add=True))
    for b in range(NBUF):
      sds[b].wait()

  plsc.subcore_barrier()

  off = sid * NSL
  @pl.when(cid == 0)
  def _():
    pltpu.sync_copy(spdeg.at[pl.ds(off, NSL)], d0_hbm.at[pl.ds(off, NSL)])
  @pl.when(cid == 1)
  def _():
    pltpu.sync_copy(spdeg.at[pl.ds(off, NSL)], d1_hbm.at[pl.ds(off, NSL)])


_deg_pass = pl.kernel(
    _deg_pass_body,
    out_type=(jax.ShapeDtypeStruct((NP,), _f32),
              jax.ShapeDtypeStruct((NP,), _f32)),
    mesh=_mesh,
    compiler_params=_cparams,
    scratch_types=[
        pltpu.VMEM((NBUF, CHUNK), _i32),
        pltpu.VMEM((CHUNK,), _f32),
        pltpu.VMEM((PCH,), _f32),
        pltpu.VMEM_SHARED((NP,), _f32),
        pltpu.SemaphoreType.DMA((1,)),
        pltpu.SemaphoreType.DMA((NBUF,)),
    ],
)


def _edge_pass_body(y_hbm, src_hbm, dst_hbm, a0_hbm, a1_hbm,
                    sidx, didx, rows, zbuf, spacc, isem, gsem, ssem):
  cid = lax.axis_index("c")
  sid = lax.axis_index("s")
  wid = _wid()

  z16 = jnp.zeros((LANES,), _f32)
  for r in range(CHUNK):
    zbuf[r, :] = z16
  zoff = sid * NSL
  @pl.loop(0, NSL // CHUNK)
  def _zero(k):
    pltpu.sync_copy(zbuf, spacc.at[pl.ds(zoff + k * CHUNK, CHUNK), :])

  plsc.subcore_barrier()

  row0, cnt = _edge_share()

  @pl.loop(0, cnt // NBUF)
  def _iter(g):
    r = row0 + g * NBUF
    di = pltpu.async_copy(src_hbm.at[pl.ds(r, NBUF), :], sidx, isem.at[0])
    dj = pltpu.async_copy(dst_hbm.at[pl.ds(r, NBUF), :], didx, isem.at[1])
    di.wait()
    dj.wait()
    gds = []
    for b in range(NBUF):
      gds.append(pltpu.async_copy(
          y_hbm.at[sidx.at[b]], rows.at[pl.ds(b * CHUNK, CHUNK), :],
          gsem.at[b]))
    sds = []
    for b in range(NBUF):
      gds[b].wait()
      sds.append(pltpu.async_copy(
          rows.at[pl.ds(b * CHUNK, CHUNK), :], spacc.at[didx.at[b]],
          ssem.at[b], add=True))
    for b in range(NBUF):
      sds[b].wait()

  plsc.subcore_barrier()

  off = sid * NSL
  @pl.when(cid == 0)
  def _():
    pltpu.sync_copy(spacc.at[pl.ds(off, NSL), :], a0_hbm.at[pl.ds(off, NSL), :])
  @pl.when(cid == 1)
  def _():
    pltpu.sync_copy(spacc.at[pl.ds(off, NSL), :], a1_hbm.at[pl.ds(off, NSL), :])


_edge_pass = pl.kernel(
    _edge_pass_body,
    out_type=(jax.ShapeDtypeStruct((NP, D), _f32),
              jax.ShapeDtypeStruct((NP, D), _f32)),
    mesh=_mesh,
    compiler_params=_cparams,
    scratch_types=[
        pltpu.VMEM((NBUF, CHUNK), _i32),
        pltpu.VMEM((NBUF, CHUNK), _i32),
        pltpu.VMEM((NBUF * CHUNK, D), _f32),
        pltpu.VMEM((CHUNK, D), _f32),
        pltpu.VMEM_SHARED((NP, D), _f32),
        pltpu.SemaphoreType.DMA((2,)),
        pltpu.SemaphoreType.DMA((NBUF,)),
        pltpu.SemaphoreType.DMA((NBUF,)),
    ],
)


def _prep_body(d0_hbm, d1_hbm, emb_hbm, arow_hbm,
               dinv16_hbm, y_hbm, out_hbm,
               v_d0, v_d1, v_emb, v_y, v_out, v_d16, v_arow):
  wid = _wid()
  pltpu.sync_copy(arow_hbm, v_arow)
  a = v_arow[...]

  @pl.loop(0, NWSL // PCH)
  def _chunk(c):
    base = wid * NWSL + c * PCH
    pltpu.sync_copy(d0_hbm.at[pl.ds(base, PCH)], v_d0)
    pltpu.sync_copy(d1_hbm.at[pl.ds(base, PCH)], v_d1)
    pltpu.sync_copy(emb_hbm.at[pl.ds(base, PCH), :], v_emb)
    for g in range(PCH // LANES):
      deg = (v_d0[pl.ds(g * LANES, LANES)]
             + v_d1[pl.ds(g * LANES, LANES)])
      degc = jnp.maximum(deg, 1.0)
      dv = jnp.where(deg > 0.0, _rsqrt16(degc), 0.0)
      for n in range(LANES):
        r = v_emb[g * LANES + n, :]
        bv = _bcast_lane(dv, n)
        v_d16[g * LANES + n, :] = bv
        v_y[g * LANES + n, :] = bv * r
        v_out[g * LANES + n, :] = a * r
    pltpu.sync_copy(v_d16, dinv16_hbm.at[pl.ds(base, PCH), :])
    pltpu.sync_copy(v_y, y_hbm.at[pl.ds(base, PCH), :])
    pltpu.sync_copy(v_out, out_hbm.at[pl.ds(base, PCH), :])


_prep = pl.kernel(
    _prep_body,
    out_type=(jax.ShapeDtypeStruct((NP, D), _f32),
              jax.ShapeDtypeStruct((NP, D), _f32),
              jax.ShapeDtypeStruct((NP, D), _f32)),
    mesh=_mesh,
    compiler_params=_cparams,
    scratch_types=[
        pltpu.VMEM((PCH,), _f32),
        pltpu.VMEM((PCH,), _f32),
        pltpu.VMEM((PCH, D), _f32),
        pltpu.VMEM((PCH, D), _f32),
        pltpu.VMEM((PCH, D), _f32),
        pltpu.VMEM((PCH, D), _f32),
        pltpu.VMEM((LANES,), _f32),
    ],
)


NPF = NP * D // 128
MBLK = 896


def _merge_tc_body(a0, a1, op, dv, al, y, out):
  x = dv[...] * (a0[...] + a1[...])
  y[...] = dv[...] * x
  out[...] = op[...] + al[...] * x


_merge = pl.pallas_call(
    _merge_tc_body,
    grid=(NPF // MBLK,),
    in_specs=[pl.BlockSpec((MBLK, 128), lambda i: (i, 0))] * 4
    + [pl.BlockSpec((1, 128), lambda i: (0, 0))],
    out_specs=[pl.BlockSpec((MBLK, 128), lambda i: (i, 0))] * 2,
    out_shape=(jax.ShapeDtypeStruct((NPF, 128), _f32),
               jax.ShapeDtypeStruct((NPF, 128), _f32)),
)


def _dot_chunk(srows, drows, v_res, cbase):
  @pl.loop(0, CHUNK // LANES, unroll=2)
  def _grp(g2):
    base16 = cbase + g2 * LANES
    acc = jnp.zeros((LANES,), _f32)
    lane = _iota16()
    for n in range(LANES):
      prod = srows[base16 + n, :] * drows[base16 + n, :]
      cs = plsc.cumsum(prod)
      tot = _bcast_lane(cs, LANES - 1)
      acc = jnp.where(lane == n, tot, acc)
    v_res[pl.ds(base16, LANES)] = acc


def _edge_dot_body(out_hbm, src_hbm, dst_hbm, res_hbm,
                   sidx, didx, srows, drows, v_res, isem, gsem):
  row0, cnt = _edge_share()

  @pl.loop(0, cnt // (2 * NBUF))
  def _iter(g):
    rA = row0 + g * (2 * NBUF)
    rB = rA + NBUF
    diA = pltpu.async_copy(
        src_hbm.at[pl.ds(rA, NBUF), :], sidx.at[pl.ds(0, NBUF), :], isem.at[0])
    djA = pltpu.async_copy(
        dst_hbm.at[pl.ds(rA, NBUF), :], didx.at[pl.ds(0, NBUF), :], isem.at[1])
    diA.wait()
    djA.wait()
    gds = []
    for b in range(NBUF):
      gds.append((
          pltpu.async_copy(
              out_hbm.at[sidx.at[b]], srows.at[pl.ds(b * CHUNK, CHUNK), :],
              gsem.at[b]),
          pltpu.async_copy(
              out_hbm.at[didx.at[b]], drows.at[pl.ds(b * CHUNK, CHUNK), :],
              gsem.at[b])))
    diB = pltpu.async_copy(
        src_hbm.at[pl.ds(rB, NBUF), :], sidx.at[pl.ds(NBUF, NBUF), :],
        isem.at[2])
    djB = pltpu.async_copy(
        dst_hbm.at[pl.ds(rB, NBUF), :], didx.at[pl.ds(NBUF, NBUF), :],
        isem.at[3])
    diB.wait()
    djB.wait()
    for b in range(NBUF):
      k = NBUF + b
      gds.append((
          pltpu.async_copy(
              out_hbm.at[sidx.at[k]], srows.at[pl.ds(k * CHUNK, CHUNK), :],
              gsem.at[k]),
          pltpu.async_copy(
              out_hbm.at[didx.at[k]], drows.at[pl.ds(k * CHUNK, CHUNK), :],
              gsem.at[k])))
    for k in range(2 * NBUF):
      gds[k][0].wait()
      gds[k][1].wait()
      _dot_chunk(srows, drows, v_res, k * CHUNK)

    pltpu.sync_copy(v_res, res_hbm.at[pl.ds(rA * CHUNK, 2 * NBUF * CHUNK)])


_edge_dot = pl.kernel(
    _edge_dot_body,
    out_type=jax.ShapeDtypeStruct((EP,), _f32),
    mesh=_mesh,
    compiler_params=_cparams,
    scratch_types=[
        pltpu.VMEM((2 * NBUF, CHUNK), _i32),
        pltpu.VMEM((2 * NBUF, CHUNK), _i32),
        pltpu.VMEM((2 * NBUF * CHUNK, D), _f32),
        pltpu.VMEM((2 * NBUF * CHUNK, D), _f32),
        pltpu.VMEM((2 * NBUF * CHUNK,), _f32),
        pltpu.SemaphoreType.DMA((4,)),
        pltpu.SemaphoreType.DMA((2 * NBUF,)),
    ],
)


@jax.jit
def kernel(edge_index, emb, alpha):
  nlayers = alpha.shape[0] - 1

  pad = jnp.full((EP - E,), N, _i32)
  src2 = jnp.concatenate([edge_index[0], pad]).reshape(EP // CHUNK, CHUNK)
  dst2 = jnp.concatenate([edge_index[1], pad]).reshape(EP // CHUNK, CHUNK)

  embp = jnp.zeros((NP, D), _f32).at[:N, :].set(emb)
  a16 = jnp.broadcast_to(alpha[:, None], (alpha.shape[0], LANES))

  d0, d1 = _deg_pass(dst2)
  dinv16, y, out = _prep(d0, d1, embp, a16[0])
  dvf = dinv16.reshape(NPF, 128)
  outf = out.reshape(NPF, 128)
  for l in range(1, nlayers + 1):
    a0, a1 = _edge_pass(y, src2, dst2)
    alf = jnp.broadcast_to(alpha[l], (1, 128))
    yf, outf = _merge(a0.reshape(NPF, 128), a1.reshape(NPF, 128),
                      outf, dvf, alf)
    y = yf.reshape(NP, D)
  res = _edge_dot(outf.reshape(NP, D), src2, dst2)
  return res[:E]
